# baseline (speedup 1.0000x reference)
import functools

import jax
import jax.numpy as jnp
from jax import lax
from jax.experimental import pallas as pl
from jax.experimental.pallas import tpu as pltpu
from jax.experimental.pallas import tpu_sc as plsc

_L = 16


def _bcast(vec, idx):
    dnums = lax.GatherDimensionNumbers(
        offset_dims=(), collapsed_slice_dims=(0,), start_index_map=(0,))
    return lax.gather(vec, idx[:, None], dnums, slice_sizes=(1,),
                      mode=lax.GatherScatterMode.PROMISE_IN_BOUNDS)


def _make_sc_kernel(N, V, D, C, NW):
    n_per_w = N // NW
    n_chunks = n_per_w // C
    mesh = plsc.VectorSubcoreMesh(core_axis_name="c", subcore_axis_name="s")

    @functools.partial(
        pl.kernel,
        out_type=jax.ShapeDtypeStruct((N, D), jnp.float32),
        mesh=mesh,
        scratch_types=[
            pltpu.VMEM((C,), jnp.int32),
            pltpu.VMEM((C,), jnp.float32),
            pltpu.VMEM((C,), jnp.float32),
            pltpu.VMEM((C,), jnp.float32),
            pltpu.VMEM((C, D), jnp.float32),
            pltpu.VMEM((C, D), jnp.float32),
            pltpu.VMEM((C, D), jnp.float32),
            pltpu.VMEM((C, D), jnp.float32),
            pltpu.SemaphoreType.DMA,
        ],
        compiler_params=pltpu.CompilerParams(use_tc_tiling_on_sc=False),
    )
    def sc_kernel(ids_hbm, wn_hbm, wz_hbm, wp_hbm, s0_hbm, s1_hbm, s2_hbm,
                  out_hbm, idx_v, p0_v, p1_v, p2_v, en_v, ez_v, ep_v, o_v,
                  sem):
        wid = lax.axis_index("s") * 2 + lax.axis_index("c")
        w_base = wid * n_per_w

        def chunk_body(c, carry):
            base = w_base + c * C
            pltpu.sync_copy(ids_hbm.at[pl.ds(base, C)], idx_v)
            cp0 = pltpu.async_copy(s0_hbm.at[idx_v], p0_v, sem)
            cp1 = pltpu.async_copy(s1_hbm.at[idx_v], p1_v, sem)
            cp2 = pltpu.async_copy(s2_hbm.at[idx_v], p2_v, sem)
            cp3 = pltpu.async_copy(wn_hbm.at[idx_v], en_v, sem)
            cp4 = pltpu.async_copy(wz_hbm.at[idx_v], ez_v, sem)
            cp5 = pltpu.async_copy(wp_hbm.at[idx_v], ep_v, sem)
            cp0.wait()
            cp1.wait()
            cp2.wait()
            cp3.wait()
            cp4.wait()
            cp5.wait()

            for i in range(C // _L):
                sl = pl.ds(i * _L, _L)
                l0 = p0_v[sl]
                l1 = p1_v[sl]
                l2 = p2_v[sl]
                m = jnp.maximum(jnp.maximum(l0, l1), l2)
                e0 = jnp.exp(l0 - m)
                e1 = jnp.exp(l1 - m)
                e2 = jnp.exp(l2 - m)
                inv = 1.0 / (e0 + e1 + e2)
                p0_v[sl] = e0 * inv
                p1_v[sl] = e1 * inv
                p2_v[sl] = e2 * inv

            def grp_body(g, carry):
                gsl = pl.ds(g * _L, _L)
                pv0 = p0_v[gsl]
                pv1 = p1_v[gsl]
                pv2 = p2_v[gsl]
                for t in range(_L):
                    j = g * _L + t
                    ts = jnp.full((_L,), t, jnp.int32)
                    pb0 = _bcast(pv0, ts)
                    pb1 = _bcast(pv1, ts)
                    pb2 = _bcast(pv2, ts)
                    for d in range(D // _L):
                        sl = pl.ds(d * _L, _L)
                        o_v[j, sl] = (pb0 * en_v[j, sl] + pb1 * ez_v[j, sl]
                                      + pb2 * ep_v[j, sl])
                return carry

            lax.fori_loop(0, C // _L, grp_body, 0)
            pltpu.sync_copy(o_v, out_hbm.at[pl.ds(base, C)])
            return carry

        lax.fori_loop(0, n_chunks, chunk_body, 0)

    return sc_kernel


@jax.jit
def kernel(input_ids, W_neg1, W_zero, W_pos1, sup_w):
    B, S = input_ids.shape
    V, D = W_neg1.shape
    N = B * S
    NW = 32
    C = 128
    ids_flat = input_ids.reshape(N).astype(jnp.int32)
    s0, s1, s2 = [sup_w[:, i] for i in range(3)]
    out = _make_sc_kernel(N, V, D, C, NW)(
        ids_flat, W_neg1, W_zero, W_pos1, s0, s1, s2)
    return out.reshape(B, S, D)
